# Initial kernel scaffold; baseline (speedup 1.0000x reference)
import jax
import jax.numpy as jnp
from jax.experimental import pallas as pl


def kernel(x, edge_index, W1, b1, W2, b2, W3, b3):
    raise NotImplementedError("write your pallas kernel here")


# speedup vs baseline: 3.4875x; 3.4875x over previous
import functools

import jax
import jax.numpy as jnp
from jax import lax
from jax.experimental import pallas as pl
from jax.experimental.pallas import tpu as pltpu
from jax.experimental.pallas import tpu_sc as plsc

N = 10000
E = 320000
D = 128
DH = D // 2

NC = 2
NS = 16
LCH = 128
KCH = 160
EPW = KCH * LCH
EPAD = NS * EPW
NPAD = 10240
RPT = NPAD // NS
GRID = 8
RB = NPAD // GRID

_mesh = plsc.VectorSubcoreMesh(
    core_axis_name="c", subcore_axis_name="s", num_cores=NC, num_subcores=NS)
_sc_params = pltpu.CompilerParams(use_tc_tiling_on_sc=False)


def _copy_rows(src_buf, dst, row0, nrows):
  for k in range(nrows // 128):
    pltpu.sync_copy(src_buf, dst.at[pl.ds(row0 + k * 128, 128)])


@functools.partial(
    pl.kernel,
    out_type=jax.ShapeDtypeStruct((NC, NPAD, DH), jnp.float32),
    mesh=_mesh,
    scratch_types=[
        pltpu.VMEM((KCH, LCH), jnp.int32),
        pltpu.VMEM((128, DH), jnp.float32),
        pltpu.VMEM((128, DH), jnp.float32),
        pltpu.VMEM_SHARED((NPAD, DH), jnp.float32),
    ],
    compiler_params=_sc_params,
)
def _deg_kernel(degidx_hbm, pat_hbm, cnt_out, idx_v, pa_v, zbuf, cnt_sh):
  cid = lax.axis_index("c")
  sid = lax.axis_index("s")

  pltpu.sync_copy(degidx_hbm.at[cid, sid], idx_v)
  pltpu.sync_copy(pat_hbm.at[0], zbuf)
  pltpu.sync_copy(pat_hbm.at[1], pa_v)

  row0 = sid * RPT
  _copy_rows(zbuf, cnt_sh, row0, RPT)
  plsc.subcore_barrier()

  def step(j, _):
    pltpu.sync_copy(pa_v, cnt_sh.at[idx_v.at[j]], add=True)
    return 0

  lax.fori_loop(0, KCH, step, 0)
  plsc.subcore_barrier()

  for k in range(RPT // 128):
    r = row0 + k * 128
    pltpu.sync_copy(cnt_sh.at[pl.ds(r, 128)], zbuf)
    pltpu.sync_copy(zbuf, cnt_out.at[cid, pl.ds(r, 128)])


@functools.partial(
    pl.kernel,
    out_type=jax.ShapeDtypeStruct((NC, NPAD, DH), jnp.float32),
    mesh=_mesh,
    scratch_types=[
        pltpu.VMEM((KCH, LCH), jnp.int32),
        pltpu.VMEM((KCH, LCH), jnp.int32),
        pltpu.VMEM((128, DH), jnp.float32),
        pltpu.VMEM((128, DH), jnp.float32),
        pltpu.VMEM_SHARED((NPAD, DH), jnp.float32),
        pltpu.SemaphoreType.DMA,
    ],
    compiler_params=_sc_params,
)
def _spmm_kernel(h_hbm, src_hbm, dst_hbm, zrow_hbm, y_out, src_v, dst_v,
                 rows_v, zbuf, y_sh, gsem):
  cid = lax.axis_index("c")
  sid = lax.axis_index("s")

  pltpu.sync_copy(src_hbm.at[sid], src_v)
  pltpu.sync_copy(dst_hbm.at[sid], dst_v)
  pltpu.sync_copy(zrow_hbm, zbuf)

  row0 = sid * RPT
  _copy_rows(zbuf, y_sh, row0, RPT)
  plsc.subcore_barrier()

  table = h_hbm.at[cid]

  def step(j, _):
    pltpu.async_copy(table.at[src_v.at[j]], rows_v, gsem).wait()
    pltpu.sync_copy(rows_v, y_sh.at[dst_v.at[j]], add=True)
    return 0

  lax.fori_loop(0, KCH, step, 0)
  plsc.subcore_barrier()

  for k in range(RPT // 128):
    r = row0 + k * 128
    pltpu.sync_copy(y_sh.at[pl.ds(r, 128)], zbuf)
    pltpu.sync_copy(zbuf, y_out.at[cid, pl.ds(r, 128)])


def _norm_col0(c_ref):
  cnt = c_ref[:, 0]
  return jnp.where(cnt > 0.0, lax.rsqrt(jnp.maximum(cnt, 1e-30)), 0.0)


def _scale_body(x_ref, cs_ref, o_ref):
  ns = _norm_col0(cs_ref)
  o_ref[...] = (x_ref[0] * ns[:, None])[None]


def _scale_x(xsplit, cs):
  return pl.pallas_call(
      _scale_body,
      grid=(GRID, NC),
      in_specs=[
          pl.BlockSpec((1, RB, DH), lambda i, c: (c, i, 0)),
          pl.BlockSpec((RB, DH), lambda i, c: (i, 0)),
      ],
      out_specs=pl.BlockSpec((1, RB, DH), lambda i, c: (c, i, 0)),
      out_shape=jax.ShapeDtypeStruct((NC, NPAD, DH), jnp.float32),
  )(xsplit, cs)


def _dense_mid_body(y0_ref, y1_ref, cs_ref, cd_ref, w_ref, b_ref, o_ref):
  nd = _norm_col0(cd_ref)
  yn = jnp.concatenate([y0_ref[...], y1_ref[...]], axis=1) * nd[:, None]
  h = jnp.dot(yn, w_ref[0], preferred_element_type=jnp.float32)
  h = jnp.maximum(h + b_ref[0], 0.0)
  ns = _norm_col0(cs_ref)
  o_ref[...] = (h * ns[:, None])[None]


def _dense_mid(y, cs, cd, w, b):
  wsplit = w.reshape(D, NC, DH).transpose(1, 0, 2)
  bsplit = b.reshape(NC, 1, DH)
  return pl.pallas_call(
      _dense_mid_body,
      grid=(GRID, NC),
      in_specs=[
          pl.BlockSpec((RB, DH), lambda i, c: (i, 0)),
          pl.BlockSpec((RB, DH), lambda i, c: (i, 0)),
          pl.BlockSpec((RB, DH), lambda i, c: (i, 0)),
          pl.BlockSpec((RB, DH), lambda i, c: (i, 0)),
          pl.BlockSpec((1, D, DH), lambda i, c: (c, 0, 0)),
          pl.BlockSpec((1, 1, DH), lambda i, c: (c, 0, 0)),
      ],
      out_specs=pl.BlockSpec((1, RB, DH), lambda i, c: (c, i, 0)),
      out_shape=jax.ShapeDtypeStruct((NC, NPAD, DH), jnp.float32),
  )(y[0], y[1], cs, cd, wsplit, bsplit)


def _dense_last_body(y0_ref, y1_ref, cd_ref, w_ref, b_ref, o_ref):
  nd = _norm_col0(cd_ref)
  yn = jnp.concatenate([y0_ref[...], y1_ref[...]], axis=1) * nd[:, None]
  h = jnp.dot(yn, w_ref[...], preferred_element_type=jnp.float32)
  o_ref[...] = jnp.maximum(h + b_ref[...], 0.0)


def _dense_last(y, cd, w, b):
  return pl.pallas_call(
      _dense_last_body,
      grid=(GRID,),
      in_specs=[
          pl.BlockSpec((RB, DH), lambda i: (i, 0)),
          pl.BlockSpec((RB, DH), lambda i: (i, 0)),
          pl.BlockSpec((RB, DH), lambda i: (i, 0)),
          pl.BlockSpec((D, D), lambda i: (0, 0)),
          pl.BlockSpec((1, D), lambda i: (0, 0)),
      ],
      out_specs=pl.BlockSpec((RB, D), lambda i: (i, 0)),
      out_shape=jax.ShapeDtypeStruct((NPAD, D), jnp.float32),
  )(y[0], y[1], cd, w, b.reshape(1, D))


def kernel(x, edge_index, W1, b1, W2, b2, W3, b3):
  pad = jnp.full((EPAD - E,), N, jnp.int32)
  srcp = jnp.concatenate([edge_index[0], pad]).reshape(NS, KCH, LCH)
  dstp = jnp.concatenate([edge_index[1], pad]).reshape(NS, KCH, LCH)
  degidx = jnp.stack([srcp, dstp])

  zrow = jnp.zeros((128, DH), jnp.float32)
  pat = jnp.stack([zrow, zrow.at[:, 0].set(1.0)])
  cnt = _deg_kernel(degidx, pat)
  cs, cd = cnt[0], cnt[1]

  xpad = jnp.pad(x, ((0, NPAD - N), (0, 0)))
  xsplit = xpad.reshape(NPAD, NC, DH).transpose(1, 0, 2)
  h = _scale_x(xsplit, cs)
  for w, b in ((W1, b1), (W2, b2)):
    y = _spmm_kernel(h, srcp, dstp, zrow)
    h = _dense_mid(y, cs, cd, w, b)
  y = _spmm_kernel(h, srcp, dstp, zrow)
  return _dense_last(y, cd, W3, b3)[:N]


# speedup vs baseline: 3.6847x; 1.0566x over previous
import functools

import jax
import jax.numpy as jnp
from jax import lax
from jax.experimental import pallas as pl
from jax.experimental.pallas import tpu as pltpu
from jax.experimental.pallas import tpu_sc as plsc

N = 10000
E = 320000
D = 128
DH = D // 2

NC = 2
NS = 16
LCH = 128
KCH = 160
EPW = KCH * LCH
EPAD = NS * EPW
NPAD = 10240
RPT = NPAD // NS
GRID = 8
RB = NPAD // GRID

_mesh = plsc.VectorSubcoreMesh(
    core_axis_name="c", subcore_axis_name="s", num_cores=NC, num_subcores=NS)
_sc_params = pltpu.CompilerParams(use_tc_tiling_on_sc=False)


def _copy_rows(src_buf, dst, row0, nrows):
  for k in range(nrows // 128):
    pltpu.sync_copy(src_buf, dst.at[pl.ds(row0 + k * 128, 128)])


@functools.partial(
    pl.kernel,
    out_type=jax.ShapeDtypeStruct((NC, NPAD, DH), jnp.float32),
    mesh=_mesh,
    scratch_types=[
        pltpu.VMEM((KCH, LCH), jnp.int32),
        pltpu.VMEM((128, DH), jnp.float32),
        pltpu.VMEM((128, DH), jnp.float32),
        pltpu.VMEM_SHARED((NPAD, DH), jnp.float32),
    ],
    compiler_params=_sc_params,
)
def _deg_kernel(degidx_hbm, pat_hbm, cnt_out, idx_v, pa_v, zbuf, cnt_sh):
  cid = lax.axis_index("c")
  sid = lax.axis_index("s")

  pltpu.sync_copy(degidx_hbm.at[cid, sid], idx_v)
  pltpu.sync_copy(pat_hbm.at[0], zbuf)
  pltpu.sync_copy(pat_hbm.at[1], pa_v)

  row0 = sid * RPT
  _copy_rows(zbuf, cnt_sh, row0, RPT)
  plsc.subcore_barrier()

  def step(j, _):
    pltpu.sync_copy(pa_v, cnt_sh.at[idx_v.at[j]], add=True)
    return 0

  lax.fori_loop(0, KCH, step, 0)
  plsc.subcore_barrier()

  for k in range(RPT // 128):
    r = row0 + k * 128
    pltpu.sync_copy(cnt_sh.at[pl.ds(r, 128)], zbuf)
    pltpu.sync_copy(zbuf, cnt_out.at[cid, pl.ds(r, 128)])


@functools.partial(
    pl.kernel,
    out_type=jax.ShapeDtypeStruct((NC, NPAD, DH), jnp.float32),
    mesh=_mesh,
    scratch_types=[
        pltpu.VMEM((KCH + 1, LCH), jnp.int32),
        pltpu.VMEM((KCH, LCH), jnp.int32),
        pltpu.VMEM((128, DH), jnp.float32),
        pltpu.VMEM((128, DH), jnp.float32),
        pltpu.VMEM((128, DH), jnp.float32),
        pltpu.VMEM_SHARED((NPAD, DH), jnp.float32),
        pltpu.SemaphoreType.DMA,
        pltpu.SemaphoreType.DMA,
    ],
    compiler_params=_sc_params,
)
def _spmm_kernel(h_hbm, src_hbm, dst_hbm, zrow_hbm, y_out, src_v, dst_v,
                 buf0, buf1, zbuf, y_sh, sem0, sem1):
  cid = lax.axis_index("c")
  sid = lax.axis_index("s")

  pltpu.sync_copy(src_hbm.at[sid], src_v)
  pltpu.sync_copy(dst_hbm.at[sid], dst_v)
  pltpu.sync_copy(zrow_hbm, zbuf)

  row0 = sid * RPT
  _copy_rows(zbuf, y_sh, row0, RPT)
  plsc.subcore_barrier()

  table = h_hbm.at[cid]

  pltpu.async_copy(table.at[src_v.at[0]], buf0, sem0)

  def step(i, _):
    j0 = 2 * i
    pltpu.make_async_copy(zrow_hbm, buf0, sem0).wait()
    pltpu.async_copy(table.at[src_v.at[j0 + 1]], buf1, sem1)
    pltpu.sync_copy(buf0, y_sh.at[dst_v.at[j0]], add=True)
    pltpu.make_async_copy(zrow_hbm, buf1, sem1).wait()
    pltpu.async_copy(table.at[src_v.at[j0 + 2]], buf0, sem0)
    pltpu.sync_copy(buf1, y_sh.at[dst_v.at[j0 + 1]], add=True)
    return 0

  lax.fori_loop(0, KCH // 2, step, 0)
  pltpu.make_async_copy(zrow_hbm, buf0, sem0).wait()
  plsc.subcore_barrier()

  for k in range(RPT // 128):
    r = row0 + k * 128
    pltpu.sync_copy(y_sh.at[pl.ds(r, 128)], zbuf)
    pltpu.sync_copy(zbuf, y_out.at[cid, pl.ds(r, 128)])


def _norm_col0(c_ref):
  cnt = c_ref[:, 0]
  return jnp.where(cnt > 0.0, lax.rsqrt(jnp.maximum(cnt, 1e-30)), 0.0)


def _scale_body(x_ref, cs_ref, o_ref):
  ns = _norm_col0(cs_ref)
  o_ref[...] = (x_ref[0] * ns[:, None])[None]


def _scale_x(xsplit, cs):
  return pl.pallas_call(
      _scale_body,
      grid=(GRID, NC),
      in_specs=[
          pl.BlockSpec((1, RB, DH), lambda i, c: (c, i, 0)),
          pl.BlockSpec((RB, DH), lambda i, c: (i, 0)),
      ],
      out_specs=pl.BlockSpec((1, RB, DH), lambda i, c: (c, i, 0)),
      out_shape=jax.ShapeDtypeStruct((NC, NPAD, DH), jnp.float32),
  )(xsplit, cs)


def _dense_mid_body(y0_ref, y1_ref, cs_ref, cd_ref, w_ref, b_ref, o_ref):
  nd = _norm_col0(cd_ref)
  yn = jnp.concatenate([y0_ref[...], y1_ref[...]], axis=1) * nd[:, None]
  h = jnp.dot(yn, w_ref[0], preferred_element_type=jnp.float32)
  h = jnp.maximum(h + b_ref[0], 0.0)
  ns = _norm_col0(cs_ref)
  o_ref[...] = (h * ns[:, None])[None]


def _dense_mid(y, cs, cd, w, b):
  wsplit = w.reshape(D, NC, DH).transpose(1, 0, 2)
  bsplit = b.reshape(NC, 1, DH)
  return pl.pallas_call(
      _dense_mid_body,
      grid=(GRID, NC),
      in_specs=[
          pl.BlockSpec((RB, DH), lambda i, c: (i, 0)),
          pl.BlockSpec((RB, DH), lambda i, c: (i, 0)),
          pl.BlockSpec((RB, DH), lambda i, c: (i, 0)),
          pl.BlockSpec((RB, DH), lambda i, c: (i, 0)),
          pl.BlockSpec((1, D, DH), lambda i, c: (c, 0, 0)),
          pl.BlockSpec((1, 1, DH), lambda i, c: (c, 0, 0)),
      ],
      out_specs=pl.BlockSpec((1, RB, DH), lambda i, c: (c, i, 0)),
      out_shape=jax.ShapeDtypeStruct((NC, NPAD, DH), jnp.float32),
  )(y[0], y[1], cs, cd, wsplit, bsplit)


def _dense_last_body(y0_ref, y1_ref, cd_ref, w_ref, b_ref, o_ref):
  nd = _norm_col0(cd_ref)
  yn = jnp.concatenate([y0_ref[...], y1_ref[...]], axis=1) * nd[:, None]
  h = jnp.dot(yn, w_ref[...], preferred_element_type=jnp.float32)
  o_ref[...] = jnp.maximum(h + b_ref[...], 0.0)


def _dense_last(y, cd, w, b):
  return pl.pallas_call(
      _dense_last_body,
      grid=(GRID,),
      in_specs=[
          pl.BlockSpec((RB, DH), lambda i: (i, 0)),
          pl.BlockSpec((RB, DH), lambda i: (i, 0)),
          pl.BlockSpec((RB, DH), lambda i: (i, 0)),
          pl.BlockSpec((D, D), lambda i: (0, 0)),
          pl.BlockSpec((1, D), lambda i: (0, 0)),
      ],
      out_specs=pl.BlockSpec((RB, D), lambda i: (i, 0)),
      out_shape=jax.ShapeDtypeStruct((NPAD, D), jnp.float32),
  )(y[0], y[1], cd, w, b.reshape(1, D))


def kernel(x, edge_index, W1, b1, W2, b2, W3, b3):
  pad = jnp.full((EPAD - E,), N, jnp.int32)
  srcp = jnp.concatenate([edge_index[0], pad]).reshape(NS, KCH, LCH)
  dstp = jnp.concatenate([edge_index[1], pad]).reshape(NS, KCH, LCH)
  degidx = jnp.stack([srcp, dstp])
  srcx = jnp.concatenate(
      [srcp, jnp.full((NS, 1, LCH), N, jnp.int32)], axis=1)

  zrow = jnp.zeros((128, DH), jnp.float32)
  pat = jnp.stack([zrow, zrow.at[:, 0].set(1.0)])
  cnt = _deg_kernel(degidx, pat)
  cs, cd = cnt[0], cnt[1]

  xpad = jnp.pad(x, ((0, NPAD - N), (0, 0)))
  xsplit = xpad.reshape(NPAD, NC, DH).transpose(1, 0, 2)
  h = _scale_x(xsplit, cs)
  for w, b in ((W1, b1), (W2, b2)):
    y = _spmm_kernel(h, srcx, dstp, zrow)
    h = _dense_mid(y, cs, cd, w, b)
  y = _spmm_kernel(h, srcx, dstp, zrow)
  return _dense_last(y, cd, W3, b3)[:N]
